# baseline (speedup 1.0000x reference)
import functools

import jax
import jax.numpy as jnp
from jax import lax
from jax.experimental import pallas as pl
from jax.experimental.pallas import tpu as pltpu
from jax.experimental.pallas import tpu_sc as plsc

VOCAB = 1000000
EMB = 64
NUM_CLASSES = 1000
B = 4096
L = 200

LP = 208
C0 = 128
C1 = LP - C0
NPAD = LP - L

NC = 2
NS = 16
NW = NC * NS
B_PER_W = B // NW


def _sc_pool(xp, table):
    mesh = plsc.VectorSubcoreMesh(core_axis_name="c", subcore_axis_name="s")

    @functools.partial(
        pl.kernel,
        out_type=jax.ShapeDtypeStruct((B, EMB), jnp.float32),
        mesh=mesh,
        compiler_params=pltpu.CompilerParams(use_tc_tiling_on_sc=False),
        scratch_types=[
            pltpu.VMEM((B_PER_W, LP), jnp.int32),
            pltpu.VMEM((B_PER_W, C0), jnp.int32),
            pltpu.VMEM((B_PER_W, C1), jnp.int32),
            pltpu.VMEM((C0, EMB), jnp.float32),
            pltpu.VMEM((C1, EMB), jnp.float32),
            pltpu.VMEM((C0, EMB), jnp.float32),
            pltpu.VMEM((C1, EMB), jnp.float32),
            pltpu.VMEM_SHARED((NS * B_PER_W, EMB), jnp.float32),
            pltpu.SemaphoreType.DMA,
            pltpu.SemaphoreType.DMA,
            pltpu.SemaphoreType.DMA,
            pltpu.SemaphoreType.DMA,
        ],
    )
    def pool(x_hbm, table_hbm, out_hbm,
             idx_v, dst0_v, dst1_v, bufa0, bufa1, bufb0, bufb1, acc_sh,
             sema0, sema1, semb0, semb1):
        s = lax.axis_index("s")
        wid = s * NC + lax.axis_index("c")
        base = wid * B_PER_W

        pltpu.sync_copy(x_hbm.at[pl.ds(base, B_PER_W)], idx_v)

        @pl.loop(0, B_PER_W)
        def _(r):
            dst = jnp.full((16,), s * B_PER_W + r, jnp.int32)
            for j in range(C0 // 16):
                dst0_v[r, pl.ds(j * 16, 16)] = dst
            for j in range(C1 // 16):
                dst1_v[r, pl.ds(j * 16, 16)] = dst

        zeros = jnp.zeros((16,), jnp.float32)

        @pl.loop(0, C0)
        def _(r):
            for j in range(EMB // 16):
                bufa0[r, pl.ds(j * 16, 16)] = zeros

        pltpu.sync_copy(bufa0, acc_sh.at[pl.ds(s * B_PER_W, B_PER_W)])

        bufs = ((bufa0, bufa1, sema0, sema1), (bufb0, bufb1, semb0, semb1))

        def start_row(r, b0, b1, s0, s1):
            cp0 = pltpu.async_copy(
                table_hbm.at[idx_v.at[r, pl.ds(0, C0)]], b0, s0)
            cp1 = pltpu.async_copy(
                table_hbm.at[idx_v.at[r, pl.ds(C0, C1)]], b1, s1)
            return cp0, cp1

        def drain_row(r, b0, b1, s0, s1):
            pltpu.make_async_copy(table_hbm.at[idx_v.at[r, pl.ds(0, C0)]],
                                  b0, s0).wait()
            pltpu.sync_copy(b0, acc_sh.at[dst0_v.at[r]], add=True)
            pltpu.make_async_copy(table_hbm.at[idx_v.at[r, pl.ds(C0, C1)]],
                                  b1, s1).wait()
            pltpu.sync_copy(b1, acc_sh.at[dst1_v.at[r]], add=True)

        start_row(0, *bufs[0])

        @pl.loop(0, B_PER_W, step=2)
        def _(r):
            start_row(r + 1, *bufs[1])
            drain_row(r, *bufs[0])

            @pl.when(r + 2 < B_PER_W)
            def _():
                start_row(r + 2, *bufs[0])

            drain_row(r + 1, *bufs[1])

        pltpu.sync_copy(acc_sh.at[pl.ds(s * B_PER_W, B_PER_W)],
                        out_hbm.at[pl.ds(base, B_PER_W)])

    return pool(xp, table)


def _tc_head(sums, fc_wt, fc_b2, t0):
    TB = 256

    def body(s_ref, w_ref, b_ref, t0_ref, o_ref):
        corr = jnp.dot(t0_ref[...], w_ref[...],
                       preferred_element_type=jnp.float32,
                       precision=lax.Precision.HIGHEST)
        o_ref[...] = (
            (jnp.dot(s_ref[...], w_ref[...],
                     preferred_element_type=jnp.float32,
                     precision=lax.Precision.HIGHEST)
             - float(NPAD) * corr) * (1.0 / L)
            + b_ref[...]
        )

    return pl.pallas_call(
        body,
        grid=(B // TB,),
        in_specs=[
            pl.BlockSpec((TB, EMB), lambda i: (i, 0)),
            pl.BlockSpec((EMB, NUM_CLASSES), lambda i: (0, 0)),
            pl.BlockSpec((1, NUM_CLASSES), lambda i: (0, 0)),
            pl.BlockSpec((1, EMB), lambda i: (0, 0)),
        ],
        out_specs=pl.BlockSpec((TB, NUM_CLASSES), lambda i: (i, 0)),
        out_shape=jax.ShapeDtypeStruct((B, NUM_CLASSES), jnp.float32),
    )(sums, fc_wt, fc_b2, t0)


def kernel(x, table, fc_w, fc_b):
    xp = jnp.pad(x.astype(jnp.int32), ((0, 0), (0, LP - L)))
    sums = _sc_pool(xp, table)
    return _tc_head(sums, fc_w.T, fc_b.reshape(1, NUM_CLASSES), table[0:1])


# speedup vs baseline: 1.0024x; 1.0024x over previous
import functools

import jax
import jax.numpy as jnp
from jax import lax
from jax.experimental import pallas as pl
from jax.experimental.pallas import tpu as pltpu
from jax.experimental.pallas import tpu_sc as plsc

VOCAB = 1000000
EMB = 64
NUM_CLASSES = 1000
B = 4096
L = 200

C0 = 128
C1 = 80
C1R = L - C0
NPAD = C0 + C1 - L

NC = 2
NS = 16
NW = NC * NS
B_PER_W = B // NW


def _sc_pool(x, table):
    mesh = plsc.VectorSubcoreMesh(core_axis_name="c", subcore_axis_name="s")

    @functools.partial(
        pl.kernel,
        out_type=jax.ShapeDtypeStruct((B, EMB), jnp.float32),
        mesh=mesh,
        compiler_params=pltpu.CompilerParams(use_tc_tiling_on_sc=False),
        scratch_types=[
            pltpu.VMEM((B_PER_W, C0), jnp.int32),
            pltpu.VMEM((B_PER_W, C1), jnp.int32),
            pltpu.VMEM((B_PER_W, C0), jnp.int32),
            pltpu.VMEM((B_PER_W, C1), jnp.int32),
            pltpu.VMEM((C0, EMB), jnp.float32),
            pltpu.VMEM((C1, EMB), jnp.float32),
            pltpu.VMEM((C0, EMB), jnp.float32),
            pltpu.VMEM((C1, EMB), jnp.float32),
            pltpu.VMEM_SHARED((NS * B_PER_W, EMB), jnp.float32),
            pltpu.SemaphoreType.DMA,
            pltpu.SemaphoreType.DMA,
            pltpu.SemaphoreType.DMA,
            pltpu.SemaphoreType.DMA,
        ],
    )
    def pool(x_hbm, table_hbm, out_hbm,
             idxa_v, idxb_v, dsta_v, dstb_v, bufa0, bufb0, bufa1, bufb1,
             acc_sh, sa0, sb0, sa1, sb1):
        s = lax.axis_index("s")
        wid = s * NC + lax.axis_index("c")
        base = wid * B_PER_W

        zeros_i = jnp.zeros((16,), jnp.int32)

        @pl.loop(0, B_PER_W)
        def _(r):
            idxb_v[r, pl.ds(C1 - 16, 16)] = zeros_i
            dst = jnp.full((16,), s * B_PER_W + r, jnp.int32)
            for j in range(C0 // 16):
                dsta_v[r, pl.ds(j * 16, 16)] = dst
            for j in range(C1 // 16):
                dstb_v[r, pl.ds(j * 16, 16)] = dst

        pltpu.sync_copy(x_hbm.at[pl.ds(base, B_PER_W), pl.ds(0, C0)], idxa_v)
        pltpu.sync_copy(x_hbm.at[pl.ds(base, B_PER_W), pl.ds(C0, C1R)],
                        idxb_v.at[:, pl.ds(0, C1R)])

        zeros = jnp.zeros((16,), jnp.float32)

        @pl.loop(0, C0)
        def _(r):
            for j in range(EMB // 16):
                bufa0[r, pl.ds(j * 16, 16)] = zeros

        pltpu.sync_copy(bufa0, acc_sh.at[pl.ds(s * B_PER_W, B_PER_W)])

        @pl.loop(0, B_PER_W, step=2)
        def _(r):
            ca0 = pltpu.async_copy(table_hbm.at[idxa_v.at[r]], bufa0, sa0)
            cb0 = pltpu.async_copy(table_hbm.at[idxb_v.at[r]], bufb0, sb0)
            ca1 = pltpu.async_copy(table_hbm.at[idxa_v.at[r + 1]], bufa1, sa1)
            cb1 = pltpu.async_copy(table_hbm.at[idxb_v.at[r + 1]], bufb1, sb1)
            ca0.wait()
            pltpu.sync_copy(bufa0, acc_sh.at[dsta_v.at[r]], add=True)
            cb0.wait()
            pltpu.sync_copy(bufb0, acc_sh.at[dstb_v.at[r]], add=True)
            ca1.wait()
            pltpu.sync_copy(bufa1, acc_sh.at[dsta_v.at[r + 1]], add=True)
            cb1.wait()
            pltpu.sync_copy(bufb1, acc_sh.at[dstb_v.at[r + 1]], add=True)

        pltpu.sync_copy(acc_sh.at[pl.ds(s * B_PER_W, B_PER_W)],
                        out_hbm.at[pl.ds(base, B_PER_W)])

    return pool(x, table)


def _tc_head(sums, fc_wt, fc_b2, t0):
    TB = 256

    def body(s_ref, w_ref, b_ref, t0_ref, o_ref):
        corr = jnp.dot(t0_ref[...], w_ref[...],
                       preferred_element_type=jnp.float32,
                       precision=lax.Precision.HIGHEST)
        o_ref[...] = (
            (jnp.dot(s_ref[...], w_ref[...],
                     preferred_element_type=jnp.float32,
                     precision=lax.Precision.HIGHEST)
             - float(NPAD) * corr) * (1.0 / L)
            + b_ref[...]
        )

    return pl.pallas_call(
        body,
        grid=(B // TB,),
        in_specs=[
            pl.BlockSpec((TB, EMB), lambda i: (i, 0)),
            pl.BlockSpec((EMB, NUM_CLASSES), lambda i: (0, 0)),
            pl.BlockSpec((1, NUM_CLASSES), lambda i: (0, 0)),
            pl.BlockSpec((1, EMB), lambda i: (0, 0)),
        ],
        out_specs=pl.BlockSpec((TB, NUM_CLASSES), lambda i: (i, 0)),
        out_shape=jax.ShapeDtypeStruct((B, NUM_CLASSES), jnp.float32),
    )(sums, fc_wt, fc_b2, t0)


def kernel(x, table, fc_w, fc_b):
    sums = _sc_pool(x.astype(jnp.int32), table)
    return _tc_head(sums, fc_w.T, fc_b.reshape(1, NUM_CLASSES), table[0:1])


# speedup vs baseline: 1.0030x; 1.0006x over previous
import functools

import jax
import jax.numpy as jnp
from jax import lax
from jax.experimental import pallas as pl
from jax.experimental.pallas import tpu as pltpu
from jax.experimental.pallas import tpu_sc as plsc

VOCAB = 1000000
EMB = 64
NUM_CLASSES = 1000
B = 4096
L = 200

C0 = 128
C1 = 80
C1R = L - C0
NPAD = C0 + C1 - L

NC = 2
NS = 16
NW = NC * NS
B_PER_W = B // NW


def _sc_pool(x, table):
    mesh = plsc.VectorSubcoreMesh(core_axis_name="c", subcore_axis_name="s")

    @functools.partial(
        pl.kernel,
        out_type=jax.ShapeDtypeStruct((B, EMB), jnp.float32),
        mesh=mesh,
        compiler_params=pltpu.CompilerParams(use_tc_tiling_on_sc=False),
        scratch_types=[
            pltpu.VMEM((B_PER_W, C0), jnp.int32),
            pltpu.VMEM((B_PER_W, C1), jnp.int32),
            pltpu.VMEM((B_PER_W, C0), jnp.int32),
            pltpu.VMEM((B_PER_W, C1), jnp.int32),
            pltpu.VMEM((C0, EMB), jnp.float32),
            pltpu.VMEM((C1, EMB), jnp.float32),
            pltpu.VMEM((C0, EMB), jnp.float32),
            pltpu.VMEM((C1, EMB), jnp.float32),
            pltpu.VMEM_SHARED((NS * B_PER_W + 8, EMB), jnp.float32),
            pltpu.SemaphoreType.DMA,
            pltpu.SemaphoreType.DMA,
            pltpu.SemaphoreType.DMA,
            pltpu.SemaphoreType.DMA,
        ],
    )
    def pool(x_hbm, table_hbm, out_hbm,
             idxa_v, idxb_v, dsta_v, dstb_v, bufa0, bufb0, bufa1, bufb1,
             acc_sh, sa0, sb0, sa1, sb1):
        s = lax.axis_index("s")
        wid = s * NC + lax.axis_index("c")
        base = wid * B_PER_W

        zeros_i = jnp.zeros((16,), jnp.int32)

        trash = jnp.full((16,), NS * B_PER_W, jnp.int32)
        lane = lax.iota(jnp.int32, 16)

        @pl.loop(0, B_PER_W)
        def _(r):
            idxb_v[r, pl.ds(C1 - 16, 16)] = zeros_i
            dst = jnp.full((16,), s * B_PER_W + r, jnp.int32)
            for j in range(C0 // 16):
                dsta_v[r, pl.ds(j * 16, 16)] = dst
            for j in range(C1 // 16):
                dstb_v[r, pl.ds(j * 16, 16)] = jnp.where(
                    lane + (j * 16) < C1R, dst, trash)

        pltpu.sync_copy(x_hbm.at[pl.ds(base, B_PER_W), pl.ds(0, C0)], idxa_v)
        pltpu.sync_copy(x_hbm.at[pl.ds(base, B_PER_W), pl.ds(C0, C1R)],
                        idxb_v.at[:, pl.ds(0, C1R)])

        zeros = jnp.zeros((16,), jnp.float32)

        @pl.loop(0, C0)
        def _(r):
            for j in range(EMB // 16):
                bufa0[r, pl.ds(j * 16, 16)] = zeros

        pltpu.sync_copy(bufa0, acc_sh.at[pl.ds(s * B_PER_W, B_PER_W)])

        @pl.loop(0, B_PER_W, step=2)
        def _(r):
            ca0 = pltpu.async_copy(table_hbm.at[idxa_v.at[r]], bufa0, sa0)
            cb0 = pltpu.async_copy(table_hbm.at[idxb_v.at[r]], bufb0, sb0)
            ca1 = pltpu.async_copy(table_hbm.at[idxa_v.at[r + 1]], bufa1, sa1)
            cb1 = pltpu.async_copy(table_hbm.at[idxb_v.at[r + 1]], bufb1, sb1)
            ca0.wait()
            pltpu.sync_copy(bufa0, acc_sh.at[dsta_v.at[r]], add=True)
            cb0.wait()
            pltpu.sync_copy(bufb0, acc_sh.at[dstb_v.at[r]], add=True)
            ca1.wait()
            pltpu.sync_copy(bufa1, acc_sh.at[dsta_v.at[r + 1]], add=True)
            cb1.wait()
            pltpu.sync_copy(bufb1, acc_sh.at[dstb_v.at[r + 1]], add=True)

        pltpu.sync_copy(acc_sh.at[pl.ds(s * B_PER_W, B_PER_W)],
                        out_hbm.at[pl.ds(base, B_PER_W)])

    return pool(x, table)


def _tc_head(sums, fc_wt, fc_b2):
    TB = 256

    def body(s_ref, w_ref, b_ref, o_ref):
        o_ref[...] = (
            jnp.dot(s_ref[...], w_ref[...],
                    preferred_element_type=jnp.float32,
                    precision=lax.Precision.HIGHEST) * (1.0 / L)
            + b_ref[...]
        )

    return pl.pallas_call(
        body,
        grid=(B // TB,),
        in_specs=[
            pl.BlockSpec((TB, EMB), lambda i: (i, 0)),
            pl.BlockSpec((EMB, NUM_CLASSES), lambda i: (0, 0)),
            pl.BlockSpec((1, NUM_CLASSES), lambda i: (0, 0)),
        ],
        out_specs=pl.BlockSpec((TB, NUM_CLASSES), lambda i: (i, 0)),
        out_shape=jax.ShapeDtypeStruct((B, NUM_CLASSES), jnp.float32),
    )(sums, fc_wt, fc_b2)


def kernel(x, table, fc_w, fc_b):
    sums = _sc_pool(x.astype(jnp.int32), table)
    return _tc_head(sums, fc_w.T, fc_b.reshape(1, NUM_CLASSES))


# speedup vs baseline: 1.5761x; 1.5714x over previous
import functools

import jax
import jax.numpy as jnp
from jax import lax
from jax.experimental import pallas as pl
from jax.experimental.pallas import tpu as pltpu
from jax.experimental.pallas import tpu_sc as plsc

VOCAB = 1000000
EMB = 64
NUM_CLASSES = 1000
B = 4096
L = 200

CHUNK = 128
NBUF = 4

NC = 2
NS = 16
NW = NC * NS
B_PER_W = B // NW
IDX_PER_W = B_PER_W * L
CHUNKS = IDX_PER_W // CHUNK


def _sc_pool(x3, seg, table):
    mesh = plsc.VectorSubcoreMesh(core_axis_name="c", subcore_axis_name="s")

    @functools.partial(
        pl.kernel,
        out_type=jax.ShapeDtypeStruct((B, EMB), jnp.float32),
        mesh=mesh,
        compiler_params=pltpu.CompilerParams(use_tc_tiling_on_sc=False),
        scratch_types=[
            pltpu.VMEM((CHUNKS, CHUNK), jnp.int32),
            pltpu.VMEM((CHUNKS, CHUNK), jnp.int32),
            pltpu.VMEM((CHUNK, EMB), jnp.float32),
            pltpu.VMEM((CHUNK, EMB), jnp.float32),
            pltpu.VMEM((CHUNK, EMB), jnp.float32),
            pltpu.VMEM((CHUNK, EMB), jnp.float32),
            pltpu.VMEM_SHARED((NS * B_PER_W, EMB), jnp.float32),
            pltpu.SemaphoreType.DMA,
            pltpu.SemaphoreType.DMA,
            pltpu.SemaphoreType.DMA,
            pltpu.SemaphoreType.DMA,
        ],
    )
    def pool(x_hbm, seg_hbm, table_hbm, out_hbm,
             idx_v, dst_v, buf0, buf1, buf2, buf3, acc_sh,
             sem0, sem1, sem2, sem3):
        s = lax.axis_index("s")
        wid = s * NC + lax.axis_index("c")
        base = wid * B_PER_W

        pltpu.sync_copy(x_hbm.at[wid], idx_v)
        pltpu.sync_copy(seg_hbm, dst_v)

        sbase = jnp.full((16,), s * B_PER_W, jnp.int32)

        @pl.loop(0, CHUNKS)
        def _(k):
            for j in range(CHUNK // 16):
                sl = pl.ds(j * 16, 16)
                dst_v[k, sl] = dst_v[k, sl] + sbase

        zeros_f = jnp.zeros((16,), jnp.float32)

        @pl.loop(0, CHUNK)
        def _(r):
            for j in range(EMB // 16):
                buf0[r, pl.ds(j * 16, 16)] = zeros_f

        pltpu.sync_copy(buf0, acc_sh.at[pl.ds(s * B_PER_W, B_PER_W)])

        @pl.loop(0, CHUNKS, step=2)
        def _(k):
            cp0 = pltpu.async_copy(table_hbm.at[idx_v.at[k]], buf0, sem0)
            cp1 = pltpu.async_copy(table_hbm.at[idx_v.at[k + 1]], buf1, sem1)
            cp0.wait()
            pltpu.sync_copy(buf0, acc_sh.at[dst_v.at[k]], add=True)
            cp1.wait()
            pltpu.sync_copy(buf1, acc_sh.at[dst_v.at[k + 1]], add=True)

        pltpu.sync_copy(acc_sh.at[pl.ds(s * B_PER_W, B_PER_W)],
                        out_hbm.at[pl.ds(base, B_PER_W)])

    return pool(x3, seg, table)


def _tc_head(sums, fc_wt, fc_b2):
    TB = 256

    def body(s_ref, w_ref, b_ref, o_ref):
        o_ref[...] = (
            jnp.dot(s_ref[...], w_ref[...],
                    preferred_element_type=jnp.float32,
                    precision=lax.Precision.HIGHEST) * (1.0 / L)
            + b_ref[...]
        )

    return pl.pallas_call(
        body,
        grid=(B // TB,),
        in_specs=[
            pl.BlockSpec((TB, EMB), lambda i: (i, 0)),
            pl.BlockSpec((EMB, NUM_CLASSES), lambda i: (0, 0)),
            pl.BlockSpec((1, NUM_CLASSES), lambda i: (0, 0)),
        ],
        out_specs=pl.BlockSpec((TB, NUM_CLASSES), lambda i: (i, 0)),
        out_shape=jax.ShapeDtypeStruct((B, NUM_CLASSES), jnp.float32),
    )(sums, fc_wt, fc_b2)


def kernel(x, table, fc_w, fc_b):
    x3 = x.astype(jnp.int32).reshape(NW, CHUNKS, CHUNK)
    seg = (jnp.arange(CHUNKS * CHUNK, dtype=jnp.int32) // L).reshape(CHUNKS, CHUNK)
    sums = _sc_pool(x3, seg, table)
    return _tc_head(sums, fc_w.T, fc_b.reshape(1, NUM_CLASSES))


# speedup vs baseline: 1.6108x; 1.0220x over previous
import functools

import jax
import jax.numpy as jnp
from jax import lax
from jax.experimental import pallas as pl
from jax.experimental.pallas import tpu as pltpu
from jax.experimental.pallas import tpu_sc as plsc

VOCAB = 1000000
EMB = 64
NUM_CLASSES = 1000
B = 4096
L = 200

CHUNK = 128
NBUF = 4

NC = 2
NS = 16
NW = NC * NS
B_PER_W = B // NW
IDX_PER_W = B_PER_W * L
CHUNKS = IDX_PER_W // CHUNK


def _sc_pool(x3, seg, table):
    mesh = plsc.VectorSubcoreMesh(core_axis_name="c", subcore_axis_name="s")

    @functools.partial(
        pl.kernel,
        out_type=jax.ShapeDtypeStruct((B, EMB), jnp.float32),
        mesh=mesh,
        compiler_params=pltpu.CompilerParams(use_tc_tiling_on_sc=False),
        scratch_types=[
            pltpu.VMEM((CHUNKS, CHUNK), jnp.int32),
            pltpu.VMEM((CHUNKS, CHUNK), jnp.int32),
            pltpu.VMEM((CHUNK, EMB), jnp.float32),
            pltpu.VMEM((CHUNK, EMB), jnp.float32),
            pltpu.VMEM((CHUNK, EMB), jnp.float32),
            pltpu.VMEM((CHUNK, EMB), jnp.float32),
            pltpu.VMEM_SHARED((NS * B_PER_W, EMB), jnp.float32),
            pltpu.SemaphoreType.DMA,
            pltpu.SemaphoreType.DMA,
            pltpu.SemaphoreType.DMA,
            pltpu.SemaphoreType.DMA,
        ],
    )
    def pool(x_hbm, seg_hbm, table_hbm, out_hbm,
             idx_v, dst_v, buf0, buf1, buf2, buf3, acc_sh,
             sem0, sem1, sem2, sem3):
        s = lax.axis_index("s")
        wid = s * NC + lax.axis_index("c")
        base = wid * B_PER_W

        pltpu.sync_copy(x_hbm.at[wid], idx_v)
        pltpu.sync_copy(seg_hbm, dst_v)

        sbase = jnp.full((16,), s * B_PER_W, jnp.int32)

        @pl.loop(0, CHUNKS)
        def _(k):
            for j in range(CHUNK // 16):
                sl = pl.ds(j * 16, 16)
                dst_v[k, sl] = dst_v[k, sl] + sbase

        zeros_f = jnp.zeros((16,), jnp.float32)

        @pl.loop(0, CHUNK)
        def _(r):
            for j in range(EMB // 16):
                buf0[r, pl.ds(j * 16, 16)] = zeros_f

        pltpu.sync_copy(buf0, acc_sh.at[pl.ds(s * B_PER_W, B_PER_W)])

        @pl.loop(0, CHUNKS, step=4)
        def _(k):
            cp0 = pltpu.async_copy(table_hbm.at[idx_v.at[k]], buf0, sem0)
            cp1 = pltpu.async_copy(table_hbm.at[idx_v.at[k + 1]], buf1, sem1)
            cp2 = pltpu.async_copy(table_hbm.at[idx_v.at[k + 2]], buf2, sem2)
            cp3 = pltpu.async_copy(table_hbm.at[idx_v.at[k + 3]], buf3, sem3)
            cp0.wait()
            pltpu.sync_copy(buf0, acc_sh.at[dst_v.at[k]], add=True)
            cp1.wait()
            pltpu.sync_copy(buf1, acc_sh.at[dst_v.at[k + 1]], add=True)
            cp2.wait()
            pltpu.sync_copy(buf2, acc_sh.at[dst_v.at[k + 2]], add=True)
            cp3.wait()
            pltpu.sync_copy(buf3, acc_sh.at[dst_v.at[k + 3]], add=True)

        pltpu.sync_copy(acc_sh.at[pl.ds(s * B_PER_W, B_PER_W)],
                        out_hbm.at[pl.ds(base, B_PER_W)])

    return pool(x3, seg, table)


def _tc_head(sums, fc_wt, fc_b2):
    TB = 256

    def body(s_ref, w_ref, b_ref, o_ref):
        o_ref[...] = (
            jnp.dot(s_ref[...], w_ref[...],
                    preferred_element_type=jnp.float32,
                    precision=lax.Precision.HIGHEST) * (1.0 / L)
            + b_ref[...]
        )

    return pl.pallas_call(
        body,
        grid=(B // TB,),
        in_specs=[
            pl.BlockSpec((TB, EMB), lambda i: (i, 0)),
            pl.BlockSpec((EMB, NUM_CLASSES), lambda i: (0, 0)),
            pl.BlockSpec((1, NUM_CLASSES), lambda i: (0, 0)),
        ],
        out_specs=pl.BlockSpec((TB, NUM_CLASSES), lambda i: (i, 0)),
        out_shape=jax.ShapeDtypeStruct((B, NUM_CLASSES), jnp.float32),
    )(sums, fc_wt, fc_b2)


def kernel(x, table, fc_w, fc_b):
    x3 = x.astype(jnp.int32).reshape(NW, CHUNKS, CHUNK)
    seg = (jnp.arange(CHUNKS * CHUNK, dtype=jnp.int32) // L).reshape(CHUNKS, CHUNK)
    sums = _sc_pool(x3, seg, table)
    return _tc_head(sums, fc_w.T, fc_b.reshape(1, NUM_CLASSES))
